# baseline (speedup 1.0000x reference)
import functools
import math

import jax
import jax.numpy as jnp
from jax import lax
from jax.experimental import pallas as pl
from jax.experimental.pallas import tpu as pltpu
from jax.experimental.pallas import tpu_sc as plsc

D_MODEL = 64
SCALE = math.sqrt(D_MODEL)

NUM_CORES = 2
NUM_SUBCORES = 16
NUM_WORKERS = NUM_CORES * NUM_SUBCORES

BLK = 128
NBUF = 2


def _emb_body(xt_hbm, lut_hbm, out_hbm, idx_all,
              in0, in1, out0, out1, sg0, sg1, sw0, sw1,
              *, seq_len):
    wid = lax.axis_index("s") * NUM_CORES + lax.axis_index("c")
    rounds = seq_len // NBUF

    ins = (in0, in1)
    outs = (out0, out1)
    sgs = (sg0, sg1)
    sws = (sw0, sw1)

    pltpu.sync_copy(xt_hbm.at[:, pl.ds(wid * BLK, BLK)], idx_all)

    iota = lax.iota(jnp.int32, 16)

    def fire_gather(s, b):
        pltpu.async_copy(lut_hbm.at[idx_all.at[s]], ins[b], sgs[b])

    def wait_gather(s, b):
        pltpu.make_async_copy(lut_hbm.at[idx_all.at[s]], ins[b], sgs[b]).wait()

    def fire_write(s, b):
        for dt in range(D_MODEL // 8):
            pltpu.async_copy(
                outs[b].at[pl.ds(dt * 8, 8)], out_hbm.at[s, dt, wid], sws[b]
            )

    def wait_write(s, b):
        for dt in range(D_MODEL // 8):
            pltpu.make_async_copy(
                outs[b].at[pl.ds(dt * 8, 8)], out_hbm.at[s, dt, wid], sws[b]
            ).wait()

    def transpose_scale(b):
        src = ins[b]
        dst = outs[b]

        def col(d, c):
            cols = jnp.full((16,), d, jnp.int32)
            for k in range(BLK // 16):
                vals = plsc.load_gather(src, [k * 16 + iota, cols])
                dst[d, pl.ds(k * 16, 16)] = vals * SCALE
            return c

        lax.fori_loop(0, D_MODEL, col, 0, unroll=2)

    for b in range(NBUF):
        fire_gather(b, b)
    for b in range(NBUF):
        wait_gather(b, b)
        transpose_scale(b)
        fire_write(b, b)
        fire_gather(b + NBUF, b)

    def round_body(g, c):
        for b in range(NBUF):
            s = g * NBUF + b
            wait_gather(s, b)
            wait_write(s - NBUF, b)
            transpose_scale(b)
            fire_write(s, b)
            fire_gather(s + NBUF, b)
        return c

    lax.fori_loop(1, rounds - 1, round_body, 0)

    for b in range(NBUF):
        s = seq_len - NBUF + b
        wait_gather(s, b)
        wait_write(s - NBUF, b)
        transpose_scale(b)
        fire_write(s, b)
    for b in range(NBUF):
        wait_write(seq_len - NBUF + b, b)


def kernel(x, lut):
    bsz, seq = x.shape
    assert bsz == NUM_WORKERS * BLK and seq % NBUF == 0
    xt = x.T

    mesh = plsc.VectorSubcoreMesh(core_axis_name="c", subcore_axis_name="s")
    run = pl.kernel(
        functools.partial(_emb_body, seq_len=seq),
        out_type=jax.ShapeDtypeStruct(
            (seq, D_MODEL // 8, NUM_WORKERS, 8, BLK), jnp.float32
        ),
        mesh=mesh,
        scratch_types=[
            pltpu.VMEM((seq, BLK), jnp.int32),
            pltpu.VMEM((BLK, D_MODEL), jnp.float32),
            pltpu.VMEM((BLK, D_MODEL), jnp.float32),
            pltpu.VMEM((D_MODEL, BLK), jnp.float32),
            pltpu.VMEM((D_MODEL, BLK), jnp.float32),
            pltpu.SemaphoreType.DMA,
            pltpu.SemaphoreType.DMA,
            pltpu.SemaphoreType.DMA,
            pltpu.SemaphoreType.DMA,
        ],
        compiler_params=pltpu.CompilerParams(
            use_tc_tiling_on_sc=False, needs_layout_passes=False
        ),
    )
    out5 = run(xt, lut)
    out = out5.transpose(2, 4, 0, 1, 3).reshape(bsz, seq, D_MODEL)
    return out
